# baseline (speedup 1.0000x reference)
import jax
import jax.numpy as jnp
from jax import lax
from jax.experimental import pallas as pl
from jax.experimental.pallas import tpu as pltpu
from jax.experimental.pallas import tpu_sc as plsc

B, H, W, C = 2, 112, 112, 96
N = B * H * W * C
OUT = B * (2 * H) * (2 * W) * C

NC = 2
NS = 16
NCHUNK = 6
ROUNDS = NCHUNK // NC
CH = OUT // NCHUNK
CPT = CH // NS
SLICE = N // NS
WIN = 7168
NWIN = SLICE // WIN
BLK = 256
NBLK = WIN // BLK
PAD = 16384


def _body(idx_hbm, val_hbm, out_hbm, idx_win, val_win, idx_blk, val_blk, acc):
    c = lax.axis_index("c")
    s = lax.axis_index("s")
    iota = lax.iota(jnp.int32, 16)
    zeros = jnp.zeros((16,), jnp.float32)

    @pl.loop(0, ROUNDS)
    def _(r):
        lo = (c * ROUNDS + r) * CH

        @pl.loop(0, WIN // 16)
        def _(i):
            val_win[pl.ds(i * 16, 16)] = zeros

        @pl.loop(0, CPT // WIN)
        def _(j):
            pltpu.sync_copy(val_win, acc.at[pl.ds(s * CPT + j * WIN, WIN)])

        plsc.subcore_barrier()

        @pl.loop(0, NWIN)
        def _(w):
            base = s * SLICE + w * WIN
            pltpu.sync_copy(idx_hbm.at[pl.ds(base, WIN)], idx_win)
            pltpu.sync_copy(val_hbm.at[pl.ds(base, WIN)], val_win)

            @pl.loop(0, WIN // 16, init_carry=jnp.int32(0), unroll=4)
            def cnt(i, cnt):
                idxv = idx_win[pl.ds(i * 16, 16)]
                valv = val_win[pl.ds(i * 16, 16)]
                rel = idxv - lo
                m = (rel >= 0) & (rel < CH)
                cs = plsc.cumsum(m.astype(jnp.int32))
                pos = cnt + cs - 1
                r_hi = pos >> 8
                r_lo = pos & (BLK - 1)
                plsc.store_scatter(idx_blk, [r_hi, r_lo], rel, mask=m)
                plsc.store_scatter(val_blk, [r_hi, r_lo], valv, mask=m)
                return cnt + cs[15]

            nblk = (cnt + BLK - 1) >> 8
            end = nblk * BLK
            npad = (end - cnt + 15) >> 4

            @pl.loop(0, npad)
            def _(i):
                p = cnt + i * 16 + iota
                dump = CH + (((p * 61) + s * 331) & (PAD - 1))
                plsc.store_scatter(
                    idx_blk, [p >> 8, p & (BLK - 1)], dump, mask=p < end
                )

            @pl.loop(0, nblk)
            def _(j):
                pltpu.sync_copy(val_blk.at[j], acc.at[idx_blk.at[j]], add=True)

        plsc.subcore_barrier()
        pltpu.sync_copy(
            acc.at[pl.ds(s * CPT, CPT)],
            out_hbm.at[pl.ds(lo + s * CPT, CPT)],
        )


@jax.jit
def kernel(input, index):
    mesh = plsc.VectorSubcoreMesh(core_axis_name="c", subcore_axis_name="s")
    run = pl.kernel(
        _body,
        out_type=jax.ShapeDtypeStruct((OUT,), jnp.float32),
        mesh=mesh,
        compiler_params=pltpu.CompilerParams(
            needs_layout_passes=False, use_tc_tiling_on_sc=False
        ),
        scratch_types=[
            pltpu.VMEM((WIN,), jnp.int32),
            pltpu.VMEM((WIN,), jnp.float32),
            pltpu.VMEM((NBLK, BLK), jnp.int32),
            pltpu.VMEM((NBLK, BLK), jnp.float32),
            pltpu.VMEM_SHARED((CH + PAD,), jnp.float32),
        ],
    )
    out = run(index.reshape(-1), input.reshape(-1))
    return out.reshape(B, 2 * H, 2 * W, C)
